# baseline (speedup 1.0000x reference)
import functools

import jax
import jax.numpy as jnp
from jax import lax
from jax.experimental import pallas as pl
from jax.experimental.pallas import tpu as pltpu
from jax.experimental.pallas import tpu_sc as plsc

N = 10000
N_Z = 20
N_CLUSTERS = 10
A = 0.5
ALPHA = 0.45
BETA = 0.25

_MM_ROWS = 2000
_BIG_ROWS = 200


def _mm_body(h_ref, w_ref, b_ref, o_ref, *, act):
    o = jnp.dot(h_ref[...], w_ref[...], preferred_element_type=jnp.float32)
    o = o + b_ref[...]
    if act:
        o = jnp.maximum(o, 0.0)
    o_ref[...] = o


def _dense(h, w, b, act):
    n, din = h.shape
    dout = w.shape[1]
    if b is None:
        b = jnp.zeros((dout,), jnp.float32)
    b2 = b.reshape(1, dout)
    return pl.pallas_call(
        functools.partial(_mm_body, act=act),
        grid=(n // _MM_ROWS,),
        in_specs=[
            pl.BlockSpec((_MM_ROWS, din), lambda i: (i, 0)),
            pl.BlockSpec((din, dout), lambda i: (0, 0)),
            pl.BlockSpec((1, dout), lambda i: (0, 0)),
        ],
        out_specs=pl.BlockSpec((_MM_ROWS, dout), lambda i: (i, 0)),
        out_shape=jax.ShapeDtypeStruct((n, dout), jnp.float32),
    )(h, w, b2)


def _tdist_body(z_ref, c_ref, o_ref):
    z = z_ref[...]
    c = c_ref[...]
    d2 = (
        jnp.sum(z * z, axis=1, keepdims=True)
        - 2.0 * lax.dot_general(z, c, (((1,), (1,)), ((), ())),
                                preferred_element_type=jnp.float32)
        + jnp.sum(c * c, axis=1)[None, :]
    )
    u = 1.0 / (1.0 + d2)
    o_ref[...] = u / jnp.sum(u, axis=1, keepdims=True)


def _tdist(z, cluster):
    n = z.shape[0]
    k, dz = cluster.shape
    return pl.pallas_call(
        _tdist_body,
        grid=(n // _MM_ROWS,),
        in_specs=[
            pl.BlockSpec((_MM_ROWS, dz), lambda i: (i, 0)),
            pl.BlockSpec((k, dz), lambda i: (0, 0)),
        ],
        out_specs=pl.BlockSpec((_MM_ROWS, k), lambda i: (i, 0)),
        out_shape=jax.ShapeDtypeStruct((n, k), jnp.float32),
    )(z, cluster)


def _big_body(adj_ref, zit_ref, zgb_ref, zgt_ref, zl_ref, eg_ref):
    adj = adj_ref[...]
    zit = zit_ref[...]
    zgt = zgt_ref[...]
    zl_ref[...] = lax.dot_general(
        adj, zit, (((1,), (1,)), ((), ())), preferred_element_type=jnp.float32)
    zgb = zgb_ref[...]
    s = jnp.dot(zgb, zgt, preferred_element_type=jnp.float32)
    eg_ref[...] = jax.nn.sigmoid(s)


def _big(adj, z_i, z_gcn):
    n = adj.shape[0]
    zit = z_i.T
    zgt = z_gcn.T
    return pl.pallas_call(
        _big_body,
        grid=(n // _BIG_ROWS,),
        in_specs=[
            pl.BlockSpec((_BIG_ROWS, n), lambda i: (i, 0)),
            pl.BlockSpec((N_Z, n), lambda i: (0, 0)),
            pl.BlockSpec((_BIG_ROWS, N_Z), lambda i: (i, 0)),
            pl.BlockSpec((N_Z, n), lambda i: (0, 0)),
        ],
        out_specs=[
            pl.BlockSpec((_BIG_ROWS, N_Z), lambda i: (i, 0)),
            pl.BlockSpec((_BIG_ROWS, n), lambda i: (i, 0)),
        ],
        out_shape=[
            jax.ShapeDtypeStruct((n, N_Z), jnp.float32),
            jax.ShapeDtypeStruct((n, n), jnp.float32),
        ],
    )(adj, zit, z_gcn, zgt)


_NC = 2
_NS = 16
_NW = _NC * _NS
_E = 160000
_EPW = _E // _NW
_K = 40
_NCHUNK = _EPW // _K
_RPT = 624
_TAIL0 = _RPT * _NS
_TAIL = N - _TAIL0


def _sc_seg_body(vals, src_hbm, dst_hbm, zeros_hbm, out,
                 src_v, dst_v, rows_v, cob, cob2, sem, shared):
    cid = lax.axis_index("c")
    sid = lax.axis_index("s")
    wid = sid * _NC + cid
    r0 = sid * _RPT
    pltpu.sync_copy(zeros_hbm, shared.at[pl.ds(r0, _RPT)])

    @pl.when(sid == _NS - 1)
    def _():
        pltpu.sync_copy(zeros_hbm.at[pl.ds(0, _TAIL)],
                        shared.at[pl.ds(_TAIL0, _TAIL)])

    plsc.subcore_barrier()

    def body(j, carry):
        off = wid * _EPW + j * _K
        pltpu.sync_copy(src_hbm.at[pl.ds(off, _K)], src_v)
        pltpu.sync_copy(dst_hbm.at[pl.ds(off, _K)], dst_v)
        pltpu.async_copy(vals.at[src_v], rows_v, sem).wait()
        pltpu.sync_copy(rows_v, shared.at[dst_v], add=True)
        return carry

    lax.fori_loop(0, _NCHUNK, body, 0)
    plsc.subcore_barrier()
    pltpu.sync_copy(shared.at[pl.ds(r0, _RPT)], cob)
    pltpu.sync_copy(cob, out.at[cid, pl.ds(r0, _RPT)])

    @pl.when(sid == _NS - 1)
    def _():
        pltpu.sync_copy(shared.at[pl.ds(_TAIL0, _TAIL)], cob2)
        pltpu.sync_copy(cob2, out.at[cid, pl.ds(_TAIL0, _TAIL)])


def _sc_seg_sum_block(vals, src, dst, w):
    zeros = jnp.zeros((_RPT, w), jnp.float32)
    mesh = plsc.VectorSubcoreMesh(core_axis_name="c", subcore_axis_name="s")
    f = pl.kernel(
        _sc_seg_body,
        out_type=jax.ShapeDtypeStruct((_NC, N, w), jnp.float32),
        mesh=mesh,
        compiler_params=pltpu.CompilerParams(use_tc_tiling_on_sc=False),
        scratch_types=[
            pltpu.VMEM((_K,), jnp.int32),
            pltpu.VMEM((_K,), jnp.int32),
            pltpu.VMEM((_K, w), jnp.float32),
            pltpu.VMEM((_RPT, w), jnp.float32),
            pltpu.VMEM((_TAIL, w), jnp.float32),
            pltpu.SemaphoreType.DMA,
            pltpu.VMEM_SHARED((N, w), jnp.float32),
        ],
    )
    part = f(vals, src, dst, zeros)
    return part[0] + part[1]


def _seg_sum_rows(s, src, dst):
    d = s.shape[1]
    if d < 32:
        s = jnp.pad(s, ((0, 0), (0, 32 - d)))
    dp = s.shape[1]
    outs = []
    for c0 in range(0, dp, 64):
        w = min(64, dp - c0)
        outs.append(_sc_seg_sum_block(s[:, c0:c0 + w], src, dst, w))
    out = outs[0] if len(outs) == 1 else jnp.concatenate(outs, axis=1)
    return out[:, :d]


def kernel(x, adj, params, edge_index):
    p = params
    n = x.shape[0]
    src = edge_index[0].astype(jnp.int32)
    dst = edge_index[1].astype(jnp.int32)
    relu = jax.nn.relu

    e1 = _dense(x, p['We1'], p['be1'], True)
    e2 = _dense(e1, p['We2'], p['be2'], True)
    e3 = _dense(e2, p['We3'], p['be3'], True)
    z_ae = _dense(e3, p['Wz'], p['bz'], False)
    d1 = _dense(z_ae, p['Wd1'], p['bd1'], True)
    d2 = _dense(d1, p['Wd2'], p['bd2'], True)
    d3 = _dense(d2, p['Wd3'], p['bd3'], True)
    x_bar = _dense(d3, p['Wxb'], p['bxb'], False)

    ones_e = jnp.ones(src.shape[0], dtype=jnp.float32)
    deg = jnp.maximum(jax.ops.segment_sum(ones_e, dst, num_segments=n), 1.0)

    def gcn(h, w, active):
        s = _dense(h, w, None, False)
        out = _seg_sum_rows(s, src, dst) / deg[:, None]
        return relu(out) if active else out

    def gt(h, wq, wk, wv, active):
        q_ = _dense(h, wq, None, False)
        k_ = _dense(h, wk, None, False)
        v_ = _dense(h, wv, None, False)
        sc = jnp.sum(q_[dst] * k_[src], axis=-1) / (q_.shape[-1] ** 0.5)
        m = jax.ops.segment_max(sc, dst, num_segments=n)
        ex = jnp.exp(sc - m[dst])
        den = jax.ops.segment_sum(ex, dst, num_segments=n)
        al = ex / (den[dst] + 1e-16)
        out = jax.ops.segment_sum(al[:, None] * v_[src], dst, num_segments=n)
        return relu(out) if active else out

    gcn_enc1 = gcn(x, p['Wg1'], True)
    gcn_enc2 = gcn((1 - A) * gcn_enc1 + A * e1, p['Wg2'], True)
    gcn_enc3 = gcn((1 - A) * gcn_enc2 + A * e2, p['Wg3'], True)
    z_gcn = gcn((1 - A) * gcn_enc3 + A * e3, p['Wg4'], False)

    g1 = gt(x, p['Wq1'], p['Wk1'], p['Wv1'], True)
    g2 = gt((1 - A) * g1 + A * e1, p['Wq2'], p['Wk2'], p['Wv2'], True)
    g3 = gt((1 - A) * g2 + A * e2, p['Wq3'], p['Wk3'], p['Wv3'], True)
    z_graph = gcn((1 - A) * g3 + A * e3, p['Wg4'], False)

    z_i = ALPHA * z_gcn + BETA * z_ae + p['gamma'] * z_graph
    z_l, edge_gcn_hat = _big(adj, z_i, z_gcn)

    gd1 = gcn(z_gcn, p['Wg5'], True)
    gd2 = gcn(gd1, p['Wg6'], True)
    gd3 = gcn(gd2, p['Wg7'], True)
    z_gcn_hat = gcn(gd3, p['Wg8'], True)

    td1 = gt(z_graph, p['Wq5'], p['Wk5'], p['Wv5'], True)
    td2 = gt(td1, p['Wq6'], p['Wk6'], p['Wv6'], True)
    td3 = gt(td2, p['Wq7'], p['Wk7'], p['Wv7'], True)
    z_graph_hat = gcn(td3, p['Wg8'], True)

    q = _tdist(z_l, p['cluster'])
    q1 = _tdist(z_ae, p['cluster'])

    return (x_bar, z_gcn_hat, z_graph_hat, edge_gcn_hat, z_ae, q, q1, z_l)


# speedup vs baseline: 1.4789x; 1.4789x over previous
import functools

import jax
import jax.numpy as jnp
from jax import lax
from jax.experimental import pallas as pl
from jax.experimental.pallas import tpu as pltpu
from jax.experimental.pallas import tpu_sc as plsc

N = 10000
N_Z = 20
N_CLUSTERS = 10
A = 0.5
ALPHA = 0.45
BETA = 0.25

_MM_ROWS = 2000
_BIG_ROWS = 200


def _mm_body(h_ref, w_ref, b_ref, o_ref, *, act):
    o = jnp.dot(h_ref[...], w_ref[...], preferred_element_type=jnp.float32)
    o = o + b_ref[...]
    if act:
        o = jnp.maximum(o, 0.0)
    o_ref[...] = o


def _dense(h, w, b, act):
    n, din = h.shape
    dout = w.shape[1]
    if b is None:
        b = jnp.zeros((dout,), jnp.float32)
    b2 = b.reshape(1, dout)
    return pl.pallas_call(
        functools.partial(_mm_body, act=act),
        grid=(n // _MM_ROWS,),
        in_specs=[
            pl.BlockSpec((_MM_ROWS, din), lambda i: (i, 0)),
            pl.BlockSpec((din, dout), lambda i: (0, 0)),
            pl.BlockSpec((1, dout), lambda i: (0, 0)),
        ],
        out_specs=pl.BlockSpec((_MM_ROWS, dout), lambda i: (i, 0)),
        out_shape=jax.ShapeDtypeStruct((n, dout), jnp.float32),
    )(h, w, b2)


def _tdist_body(z_ref, c_ref, o_ref):
    z = z_ref[...]
    c = c_ref[...]
    d2 = (
        jnp.sum(z * z, axis=1, keepdims=True)
        - 2.0 * lax.dot_general(z, c, (((1,), (1,)), ((), ())),
                                preferred_element_type=jnp.float32)
        + jnp.sum(c * c, axis=1)[None, :]
    )
    u = 1.0 / (1.0 + d2)
    o_ref[...] = u / jnp.sum(u, axis=1, keepdims=True)


def _tdist(z, cluster):
    n = z.shape[0]
    k, dz = cluster.shape
    return pl.pallas_call(
        _tdist_body,
        grid=(n // _MM_ROWS,),
        in_specs=[
            pl.BlockSpec((_MM_ROWS, dz), lambda i: (i, 0)),
            pl.BlockSpec((k, dz), lambda i: (0, 0)),
        ],
        out_specs=pl.BlockSpec((_MM_ROWS, k), lambda i: (i, 0)),
        out_shape=jax.ShapeDtypeStruct((n, k), jnp.float32),
    )(z, cluster)


def _big_body(adj_ref, zit_ref, zgb_ref, zgt_ref, zl_ref, eg_ref):
    adj = adj_ref[...]
    zit = zit_ref[...]
    zgt = zgt_ref[...]
    zl_ref[...] = lax.dot_general(
        adj, zit, (((1,), (1,)), ((), ())), preferred_element_type=jnp.float32)
    zgb = zgb_ref[...]
    s = jnp.dot(zgb, zgt, preferred_element_type=jnp.float32)
    eg_ref[...] = jax.nn.sigmoid(s)


def _big(adj, z_i, z_gcn):
    n = adj.shape[0]
    zit = z_i.T
    zgt = z_gcn.T
    return pl.pallas_call(
        _big_body,
        grid=(n // _BIG_ROWS,),
        in_specs=[
            pl.BlockSpec((_BIG_ROWS, n), lambda i: (i, 0)),
            pl.BlockSpec((N_Z, n), lambda i: (0, 0)),
            pl.BlockSpec((_BIG_ROWS, N_Z), lambda i: (i, 0)),
            pl.BlockSpec((N_Z, n), lambda i: (0, 0)),
        ],
        out_specs=[
            pl.BlockSpec((_BIG_ROWS, N_Z), lambda i: (i, 0)),
            pl.BlockSpec((_BIG_ROWS, n), lambda i: (i, 0)),
        ],
        out_shape=[
            jax.ShapeDtypeStruct((n, N_Z), jnp.float32),
            jax.ShapeDtypeStruct((n, n), jnp.float32),
        ],
    )(adj, zit, z_gcn, zgt)


_NC = 2
_NS = 16
_NW = _NC * _NS
_E = 160000
_EPW = _E // _NW
_K2 = 128
_NCH = 40
_EPW2 = _NCH * _K2
_EP = _NW * _EPW2
_NP = 10008
_RPT = 624
_TAIL0 = _RPT * _NS
_TAIL = _NP - _TAIL0

_MESH = dict(core_axis_name="c", subcore_axis_name="s")


def _zero_shared(zeros_hbm, shared, sid):
    r0 = sid * _RPT
    pltpu.sync_copy(zeros_hbm, shared.at[pl.ds(r0, _RPT)])

    @pl.when(sid == _NS - 1)
    def _():
        pltpu.sync_copy(zeros_hbm.at[pl.ds(0, _TAIL)],
                        shared.at[pl.ds(_TAIL0, _TAIL)])


def _copy_out(out, shared, cob, cob2, cid, sid):
    r0 = sid * _RPT
    pltpu.sync_copy(shared.at[pl.ds(r0, _RPT)], cob)
    pltpu.sync_copy(cob, out.at[cid, pl.ds(r0, _RPT)])

    @pl.when(sid == _NS - 1)
    def _():
        pltpu.sync_copy(shared.at[pl.ds(_TAIL0, _TAIL)], cob2)
        pltpu.sync_copy(cob2, out.at[cid, pl.ds(_TAIL0, _TAIL)])


def _sc_seg_body(vals, src_hbm, dst_hbm, zeros_hbm, out,
                 src_v, dst_v, rows_v, cob, cob2, sem, shared):
    cid = lax.axis_index("c")
    sid = lax.axis_index("s")
    wid = sid * _NC + cid
    pltpu.sync_copy(src_hbm.at[wid], src_v)
    pltpu.sync_copy(dst_hbm.at[wid], dst_v)
    _zero_shared(zeros_hbm, shared, sid)
    plsc.subcore_barrier()

    def body(j, carry):
        pltpu.async_copy(vals.at[src_v.at[j]], rows_v, sem).wait()
        pltpu.sync_copy(rows_v, shared.at[dst_v.at[j]], add=True)
        return carry

    lax.fori_loop(0, _NCH, body, 0)
    plsc.subcore_barrier()
    _copy_out(out, shared, cob, cob2, cid, sid)


def _sc_lin_body(u, dst_hbm, zeros_hbm, out,
                 dst_v, rows_v, cob, cob2, shared):
    cid = lax.axis_index("c")
    sid = lax.axis_index("s")
    wid = sid * _NC + cid
    pltpu.sync_copy(dst_hbm.at[wid], dst_v)
    _zero_shared(zeros_hbm, shared, sid)
    plsc.subcore_barrier()

    def body(j, carry):
        off = wid * _EPW2 + j * _K2
        pltpu.sync_copy(u.at[pl.ds(off, _K2)], rows_v)
        pltpu.sync_copy(rows_v, shared.at[dst_v.at[j]], add=True)
        return carry

    lax.fori_loop(0, _NCH, body, 0)
    plsc.subcore_barrier()
    _copy_out(out, shared, cob, cob2, cid, sid)


def _sc_gather_body(vals, src_hbm, out, src_v, rows_v, sem):
    cid = lax.axis_index("c")
    sid = lax.axis_index("s")
    wid = sid * _NC + cid
    pltpu.sync_copy(src_hbm.at[wid], src_v)

    def body(j, carry):
        off = wid * _EPW2 + j * _K2
        pltpu.async_copy(vals.at[src_v.at[j]], rows_v, sem).wait()
        pltpu.sync_copy(rows_v, out.at[pl.ds(off, _K2)])
        return carry

    lax.fori_loop(0, _NCH, body, 0)


def _sc_seg_sum_block(vals, src2, dst2, w):
    zeros = jnp.zeros((_RPT, w), jnp.float32)
    f = pl.kernel(
        _sc_seg_body,
        out_type=jax.ShapeDtypeStruct((_NC, _NP, w), jnp.float32),
        mesh=plsc.VectorSubcoreMesh(**_MESH),
        compiler_params=pltpu.CompilerParams(use_tc_tiling_on_sc=False),
        scratch_types=[
            pltpu.VMEM((_NCH, _K2), jnp.int32),
            pltpu.VMEM((_NCH, _K2), jnp.int32),
            pltpu.VMEM((_K2, w), jnp.float32),
            pltpu.VMEM((_RPT, w), jnp.float32),
            pltpu.VMEM((_TAIL, w), jnp.float32),
            pltpu.SemaphoreType.DMA,
            pltpu.VMEM_SHARED((_NP, w), jnp.float32),
        ],
    )
    part = f(vals, src2, dst2, zeros)
    return part[0] + part[1]


def _sc_lin_scatter_block(u, dst2, w):
    zeros = jnp.zeros((_RPT, w), jnp.float32)
    f = pl.kernel(
        _sc_lin_body,
        out_type=jax.ShapeDtypeStruct((_NC, _NP, w), jnp.float32),
        mesh=plsc.VectorSubcoreMesh(**_MESH),
        compiler_params=pltpu.CompilerParams(use_tc_tiling_on_sc=False),
        scratch_types=[
            pltpu.VMEM((_NCH, _K2), jnp.int32),
            pltpu.VMEM((_K2, w), jnp.float32),
            pltpu.VMEM((_RPT, w), jnp.float32),
            pltpu.VMEM((_TAIL, w), jnp.float32),
            pltpu.VMEM_SHARED((_NP, w), jnp.float32),
        ],
    )
    part = f(u, dst2, zeros)
    return part[0] + part[1]


def _sc_gather(vals, idx2):
    d = vals.shape[1]
    f = pl.kernel(
        _sc_gather_body,
        out_type=jax.ShapeDtypeStruct((_EP, d), jnp.float32),
        mesh=plsc.VectorSubcoreMesh(**_MESH),
        scratch_types=[
            pltpu.VMEM((_NCH, _K2), jnp.int32),
            pltpu.VMEM((_K2, d), jnp.float32),
            pltpu.SemaphoreType.DMA,
        ],
    )
    return f(vals, idx2)


def _seg_sum_rows(s, src2, dst2):
    d = s.shape[1]
    if d < 32:
        s = jnp.pad(s, ((0, 0), (0, 32 - d)))
    dp = s.shape[1]
    outs = []
    for c0 in range(0, dp, 64):
        w = min(64, dp - c0)
        outs.append(_sc_seg_sum_block(s[:, c0:c0 + w], src2, dst2, w))
    out = outs[0] if len(outs) == 1 else jnp.concatenate(outs, axis=1)
    return out[:N, :d]


def _seg_sum_edges(u, dst2):
    d = u.shape[1]
    outs = []
    for c0 in range(0, d, 64):
        w = min(64, d - c0)
        outs.append(_sc_lin_scatter_block(u[:, c0:c0 + w], dst2, w))
    out = outs[0] if len(outs) == 1 else jnp.concatenate(outs, axis=1)
    return out[:N]


def kernel(x, adj, params, edge_index):
    p = params
    n = x.shape[0]
    src = edge_index[0].astype(jnp.int32)
    dst = edge_index[1].astype(jnp.int32)
    relu = jax.nn.relu

    e_pad = _EP - src.shape[0]
    src2 = jnp.pad(src, (0, e_pad)).reshape(_NW, _NCH, _K2)
    dst2 = jnp.pad(dst, (0, e_pad), constant_values=N).reshape(_NW, _NCH, _K2)

    e1 = _dense(x, p['We1'], p['be1'], True)
    e2 = _dense(e1, p['We2'], p['be2'], True)
    e3 = _dense(e2, p['We3'], p['be3'], True)
    z_ae = _dense(e3, p['Wz'], p['bz'], False)
    d1 = _dense(z_ae, p['Wd1'], p['bd1'], True)
    d2 = _dense(d1, p['Wd2'], p['bd2'], True)
    d3 = _dense(d2, p['Wd3'], p['bd3'], True)
    x_bar = _dense(d3, p['Wxb'], p['bxb'], False)

    ones_u = jnp.ones((_EP, 64), jnp.float32)
    deg = jnp.maximum(_seg_sum_edges(ones_u, dst2)[:, 0], 1.0)

    def gcn(h, w, active):
        s = _dense(h, w, None, False)
        out = _seg_sum_rows(s, src2, dst2) / deg[:, None]
        return relu(out) if active else out

    def gt(h, wq, wk, wv, active):
        q_ = _dense(h, wq, None, False)
        k_ = _dense(h, wk, None, False)
        v_ = _dense(h, wv, None, False)
        qg = _sc_gather(q_, dst2)
        kg = _sc_gather(k_, src2)
        vg = _sc_gather(v_, src2)
        sc = jnp.sum(qg * kg, axis=-1) / (q_.shape[-1] ** 0.5)
        m = jnp.max(sc[:src.shape[0]])
        ex = jnp.exp(sc - m)
        den = _seg_sum_edges(jnp.broadcast_to(ex[:, None], (_EP, 64)), dst2)[:, 0]
        u = ex[:, None] * vg
        out = _seg_sum_edges(u, dst2) / (den[:, None] + 1e-16)
        return relu(out) if active else out

    gcn_enc1 = gcn(x, p['Wg1'], True)
    gcn_enc2 = gcn((1 - A) * gcn_enc1 + A * e1, p['Wg2'], True)
    gcn_enc3 = gcn((1 - A) * gcn_enc2 + A * e2, p['Wg3'], True)
    z_gcn = gcn((1 - A) * gcn_enc3 + A * e3, p['Wg4'], False)

    g1 = gt(x, p['Wq1'], p['Wk1'], p['Wv1'], True)
    g2 = gt((1 - A) * g1 + A * e1, p['Wq2'], p['Wk2'], p['Wv2'], True)
    g3 = gt((1 - A) * g2 + A * e2, p['Wq3'], p['Wk3'], p['Wv3'], True)
    z_graph = gcn((1 - A) * g3 + A * e3, p['Wg4'], False)

    z_i = ALPHA * z_gcn + BETA * z_ae + p['gamma'] * z_graph
    z_l, edge_gcn_hat = _big(adj, z_i, z_gcn)

    gd1 = gcn(z_gcn, p['Wg5'], True)
    gd2 = gcn(gd1, p['Wg6'], True)
    gd3 = gcn(gd2, p['Wg7'], True)
    z_gcn_hat = gcn(gd3, p['Wg8'], True)

    td1 = gt(z_graph, p['Wq5'], p['Wk5'], p['Wv5'], True)
    td2 = gt(td1, p['Wq6'], p['Wk6'], p['Wv6'], True)
    td3 = gt(td2, p['Wq7'], p['Wk7'], p['Wv7'], True)
    z_graph_hat = gcn(td3, p['Wg8'], True)

    q = _tdist(z_l, p['cluster'])
    q1 = _tdist(z_ae, p['cluster'])

    return (x_bar, z_gcn_hat, z_graph_hat, edge_gcn_hat, z_ae, q, q1, z_l)


# speedup vs baseline: 1.5909x; 1.0757x over previous
import functools

import jax
import jax.numpy as jnp
from jax import lax
from jax.experimental import pallas as pl
from jax.experimental.pallas import tpu as pltpu
from jax.experimental.pallas import tpu_sc as plsc

N = 10000
N_Z = 20
N_CLUSTERS = 10
A = 0.5
ALPHA = 0.45
BETA = 0.25

_MM_ROWS = 2000
_BIG_ROWS = 200


def _mm_body(h_ref, w_ref, b_ref, o_ref, *, act):
    o = jnp.dot(h_ref[...], w_ref[...], preferred_element_type=jnp.float32)
    o = o + b_ref[...]
    if act:
        o = jnp.maximum(o, 0.0)
    o_ref[...] = o


def _dense(h, w, b, act):
    n, din = h.shape
    dout = w.shape[1]
    if b is None:
        b = jnp.zeros((dout,), jnp.float32)
    b2 = b.reshape(1, dout)
    return pl.pallas_call(
        functools.partial(_mm_body, act=act),
        grid=(n // _MM_ROWS,),
        in_specs=[
            pl.BlockSpec((_MM_ROWS, din), lambda i: (i, 0)),
            pl.BlockSpec((din, dout), lambda i: (0, 0)),
            pl.BlockSpec((1, dout), lambda i: (0, 0)),
        ],
        out_specs=pl.BlockSpec((_MM_ROWS, dout), lambda i: (i, 0)),
        out_shape=jax.ShapeDtypeStruct((n, dout), jnp.float32),
    )(h, w, b2)


def _tdist_body(z_ref, c_ref, o_ref):
    z = z_ref[...]
    c = c_ref[...]
    d2 = (
        jnp.sum(z * z, axis=1, keepdims=True)
        - 2.0 * lax.dot_general(z, c, (((1,), (1,)), ((), ())),
                                preferred_element_type=jnp.float32)
        + jnp.sum(c * c, axis=1)[None, :]
    )
    u = 1.0 / (1.0 + d2)
    o_ref[...] = u / jnp.sum(u, axis=1, keepdims=True)


def _tdist(z, cluster):
    n = z.shape[0]
    k, dz = cluster.shape
    return pl.pallas_call(
        _tdist_body,
        grid=(n // _MM_ROWS,),
        in_specs=[
            pl.BlockSpec((_MM_ROWS, dz), lambda i: (i, 0)),
            pl.BlockSpec((k, dz), lambda i: (0, 0)),
        ],
        out_specs=pl.BlockSpec((_MM_ROWS, k), lambda i: (i, 0)),
        out_shape=jax.ShapeDtypeStruct((n, k), jnp.float32),
    )(z, cluster)


def _big_body(adj_ref, zit_ref, zgb_ref, zgt_ref, zl_ref, eg_ref):
    adj = adj_ref[...]
    zit = zit_ref[...]
    zgt = zgt_ref[...]
    zl_ref[...] = lax.dot_general(
        adj, zit, (((1,), (1,)), ((), ())), preferred_element_type=jnp.float32)
    zgb = zgb_ref[...]
    s = jnp.dot(zgb, zgt, preferred_element_type=jnp.float32)
    eg_ref[...] = jax.nn.sigmoid(s)


def _big(adj, z_i, z_gcn):
    n = adj.shape[0]
    zit = z_i.T
    zgt = z_gcn.T
    return pl.pallas_call(
        _big_body,
        grid=(n // _BIG_ROWS,),
        in_specs=[
            pl.BlockSpec((_BIG_ROWS, n), lambda i: (i, 0)),
            pl.BlockSpec((N_Z, n), lambda i: (0, 0)),
            pl.BlockSpec((_BIG_ROWS, N_Z), lambda i: (i, 0)),
            pl.BlockSpec((N_Z, n), lambda i: (0, 0)),
        ],
        out_specs=[
            pl.BlockSpec((_BIG_ROWS, N_Z), lambda i: (i, 0)),
            pl.BlockSpec((_BIG_ROWS, n), lambda i: (i, 0)),
        ],
        out_shape=[
            jax.ShapeDtypeStruct((n, N_Z), jnp.float32),
            jax.ShapeDtypeStruct((n, n), jnp.float32),
        ],
    )(adj, zit, z_gcn, zgt)


_NC = 2
_NS = 16
_NW = _NC * _NS
_E = 160000
_EPW = _E // _NW
_K2 = 128
_NCH = 40
_EPW2 = _NCH * _K2
_EP = _NW * _EPW2
_NP = 10008
_RPT = 624
_TAIL0 = _RPT * _NS
_TAIL = _NP - _TAIL0

_MESH = dict(core_axis_name="c", subcore_axis_name="s")


def _zero_shared(zeros_hbm, shared, sid):
    r0 = sid * _RPT
    pltpu.sync_copy(zeros_hbm, shared.at[pl.ds(r0, _RPT)])

    @pl.when(sid == _NS - 1)
    def _():
        pltpu.sync_copy(zeros_hbm.at[pl.ds(0, _TAIL)],
                        shared.at[pl.ds(_TAIL0, _TAIL)])


def _copy_out(out, shared, cob, cob2, cid, sid):
    r0 = sid * _RPT
    pltpu.sync_copy(shared.at[pl.ds(r0, _RPT)], cob)
    pltpu.sync_copy(cob, out.at[cid, pl.ds(r0, _RPT)])

    @pl.when(sid == _NS - 1)
    def _():
        pltpu.sync_copy(shared.at[pl.ds(_TAIL0, _TAIL)], cob2)
        pltpu.sync_copy(cob2, out.at[cid, pl.ds(_TAIL0, _TAIL)])


def _sc_seg_body(vals, src_hbm, dst_hbm, zeros_hbm, out,
                 src_v, dst_v, rows_v, cob, cob2, sem, shared):
    cid = lax.axis_index("c")
    sid = lax.axis_index("s")
    wid = sid * _NC + cid
    pltpu.sync_copy(src_hbm.at[wid], src_v)
    pltpu.sync_copy(dst_hbm.at[wid], dst_v)
    _zero_shared(zeros_hbm, shared, sid)
    plsc.subcore_barrier()

    def body(j, carry):
        pltpu.async_copy(vals.at[src_v.at[j]], rows_v, sem).wait()
        pltpu.sync_copy(rows_v, shared.at[dst_v.at[j]], add=True)
        return carry

    lax.fori_loop(0, _NCH, body, 0)
    plsc.subcore_barrier()
    _copy_out(out, shared, cob, cob2, cid, sid)


def _sc_lin_body(u, dst_hbm, zeros_hbm, out,
                 dst_v, rows_v, cob, cob2, shared):
    cid = lax.axis_index("c")
    sid = lax.axis_index("s")
    wid = sid * _NC + cid
    pltpu.sync_copy(dst_hbm.at[wid], dst_v)
    _zero_shared(zeros_hbm, shared, sid)
    plsc.subcore_barrier()

    def body(j, carry):
        off = wid * _EPW2 + j * _K2
        pltpu.sync_copy(u.at[pl.ds(off, _K2)], rows_v)
        pltpu.sync_copy(rows_v, shared.at[dst_v.at[j]], add=True)
        return carry

    lax.fori_loop(0, _NCH, body, 0)
    plsc.subcore_barrier()
    _copy_out(out, shared, cob, cob2, cid, sid)


def _sc_wseg_body(vals, wts, src_hbm, dst_hbm, zeros_hbm, out,
                  src_v, dst_v, w_v, rows_v, cob, cob2, sem, shared):
    cid = lax.axis_index("c")
    sid = lax.axis_index("s")
    wid = sid * _NC + cid
    pltpu.sync_copy(src_hbm.at[wid], src_v)
    pltpu.sync_copy(dst_hbm.at[wid], dst_v)
    _zero_shared(zeros_hbm, shared, sid)
    plsc.subcore_barrier()

    def body(j, carry):
        off = wid * _EPW2 + j * _K2
        pltpu.async_copy(vals.at[src_v.at[j]], rows_v, sem).wait()
        pltpu.sync_copy(wts.at[pl.ds(off, _K2)], w_v)
        for g in range(_K2 // 16):
            w16 = w_v[pl.ds(g * 16, 16)]
            for i0 in range(16):
                i = g * 16 + i0
                wi = w16[i0]
                for c in range(4):
                    sl = pl.ds(c * 16, 16)
                    rows_v[i, sl] = rows_v[i, sl] * wi
        pltpu.sync_copy(rows_v, shared.at[dst_v.at[j]], add=True)
        return carry

    lax.fori_loop(0, _NCH, body, 0)
    plsc.subcore_barrier()
    _copy_out(out, shared, cob, cob2, cid, sid)


def _sc_wseg_block(vals, wts, src2, dst2, w):
    zeros = jnp.zeros((_RPT, w), jnp.float32)
    f = pl.kernel(
        _sc_wseg_body,
        out_type=jax.ShapeDtypeStruct((_NC, _NP, w), jnp.float32),
        mesh=plsc.VectorSubcoreMesh(**_MESH),
        compiler_params=pltpu.CompilerParams(use_tc_tiling_on_sc=False),
        scratch_types=[
            pltpu.VMEM((_NCH, _K2), jnp.int32),
            pltpu.VMEM((_NCH, _K2), jnp.int32),
            pltpu.VMEM((_K2,), jnp.float32),
            pltpu.VMEM((_K2, w), jnp.float32),
            pltpu.VMEM((_RPT, w), jnp.float32),
            pltpu.VMEM((_TAIL, w), jnp.float32),
            pltpu.SemaphoreType.DMA,
            pltpu.VMEM_SHARED((_NP, w), jnp.float32),
        ],
    )
    part = f(vals, wts, src2, dst2, zeros)
    return part[0] + part[1]


def _wseg_sum_rows(s, wts, src2, dst2):
    d = s.shape[1]
    outs = []
    for c0 in range(0, d, 64):
        w = min(64, d - c0)
        outs.append(_sc_wseg_block(s[:, c0:c0 + w], wts, src2, dst2, w))
    out = outs[0] if len(outs) == 1 else jnp.concatenate(outs, axis=1)
    return out[:N]


def _sc_gather_body(vals, src_hbm, out, src_v, rows_v, sem):
    cid = lax.axis_index("c")
    sid = lax.axis_index("s")
    wid = sid * _NC + cid
    pltpu.sync_copy(src_hbm.at[wid], src_v)

    def body(j, carry):
        off = wid * _EPW2 + j * _K2
        pltpu.async_copy(vals.at[src_v.at[j]], rows_v, sem).wait()
        pltpu.sync_copy(rows_v, out.at[pl.ds(off, _K2)])
        return carry

    lax.fori_loop(0, _NCH, body, 0)


def _sc_seg_sum_block(vals, src2, dst2, w):
    zeros = jnp.zeros((_RPT, w), jnp.float32)
    f = pl.kernel(
        _sc_seg_body,
        out_type=jax.ShapeDtypeStruct((_NC, _NP, w), jnp.float32),
        mesh=plsc.VectorSubcoreMesh(**_MESH),
        compiler_params=pltpu.CompilerParams(use_tc_tiling_on_sc=False),
        scratch_types=[
            pltpu.VMEM((_NCH, _K2), jnp.int32),
            pltpu.VMEM((_NCH, _K2), jnp.int32),
            pltpu.VMEM((_K2, w), jnp.float32),
            pltpu.VMEM((_RPT, w), jnp.float32),
            pltpu.VMEM((_TAIL, w), jnp.float32),
            pltpu.SemaphoreType.DMA,
            pltpu.VMEM_SHARED((_NP, w), jnp.float32),
        ],
    )
    part = f(vals, src2, dst2, zeros)
    return part[0] + part[1]


def _sc_lin_scatter_block(u, dst2, w):
    zeros = jnp.zeros((_RPT, w), jnp.float32)
    f = pl.kernel(
        _sc_lin_body,
        out_type=jax.ShapeDtypeStruct((_NC, _NP, w), jnp.float32),
        mesh=plsc.VectorSubcoreMesh(**_MESH),
        compiler_params=pltpu.CompilerParams(use_tc_tiling_on_sc=False),
        scratch_types=[
            pltpu.VMEM((_NCH, _K2), jnp.int32),
            pltpu.VMEM((_K2, w), jnp.float32),
            pltpu.VMEM((_RPT, w), jnp.float32),
            pltpu.VMEM((_TAIL, w), jnp.float32),
            pltpu.VMEM_SHARED((_NP, w), jnp.float32),
        ],
    )
    part = f(u, dst2, zeros)
    return part[0] + part[1]


def _sc_gather(vals, idx2):
    d = vals.shape[1]
    f = pl.kernel(
        _sc_gather_body,
        out_type=jax.ShapeDtypeStruct((_EP, d), jnp.float32),
        mesh=plsc.VectorSubcoreMesh(**_MESH),
        scratch_types=[
            pltpu.VMEM((_NCH, _K2), jnp.int32),
            pltpu.VMEM((_K2, d), jnp.float32),
            pltpu.SemaphoreType.DMA,
        ],
    )
    return f(vals, idx2)


def _seg_sum_rows(s, src2, dst2):
    d = s.shape[1]
    if d < 32:
        s = jnp.pad(s, ((0, 0), (0, 32 - d)))
    dp = s.shape[1]
    outs = []
    for c0 in range(0, dp, 64):
        w = min(64, dp - c0)
        outs.append(_sc_seg_sum_block(s[:, c0:c0 + w], src2, dst2, w))
    out = outs[0] if len(outs) == 1 else jnp.concatenate(outs, axis=1)
    return out[:N, :d]


def _seg_sum_edges(u, dst2):
    d = u.shape[1]
    outs = []
    for c0 in range(0, d, 64):
        w = min(64, d - c0)
        outs.append(_sc_lin_scatter_block(u[:, c0:c0 + w], dst2, w))
    out = outs[0] if len(outs) == 1 else jnp.concatenate(outs, axis=1)
    return out[:N]


def kernel(x, adj, params, edge_index):
    p = params
    n = x.shape[0]
    src = edge_index[0].astype(jnp.int32)
    dst = edge_index[1].astype(jnp.int32)
    relu = jax.nn.relu

    e_pad = _EP - src.shape[0]
    src2 = jnp.pad(src, (0, e_pad)).reshape(_NW, _NCH, _K2)
    dst2 = jnp.pad(dst, (0, e_pad), constant_values=N).reshape(_NW, _NCH, _K2)

    e1 = _dense(x, p['We1'], p['be1'], True)
    e2 = _dense(e1, p['We2'], p['be2'], True)
    e3 = _dense(e2, p['We3'], p['be3'], True)
    z_ae = _dense(e3, p['Wz'], p['bz'], False)
    d1 = _dense(z_ae, p['Wd1'], p['bd1'], True)
    d2 = _dense(d1, p['Wd2'], p['bd2'], True)
    d3 = _dense(d2, p['Wd3'], p['bd3'], True)
    x_bar = _dense(d3, p['Wxb'], p['bxb'], False)

    ones_u = jnp.ones((_EP, 64), jnp.float32)
    ones_n = jnp.ones((n, 64), jnp.float32)
    deg = jnp.maximum(_seg_sum_edges(ones_u, dst2)[:, 0], 1.0)

    def gcn(h, w, active):
        s = _dense(h, w, None, False)
        out = _seg_sum_rows(s, src2, dst2) / deg[:, None]
        return relu(out) if active else out

    def gt(h, wq, wk, wv, active):
        q_ = _dense(h, wq, None, False)
        k_ = _dense(h, wk, None, False)
        v_ = _dense(h, wv, None, False)
        qg = _sc_gather(q_, dst2)
        kg = _sc_gather(k_, src2)
        sc = jnp.sum(qg * kg, axis=-1) / (q_.shape[-1] ** 0.5)
        m = jnp.max(sc[:src.shape[0]])
        ex = jnp.exp(sc - m)
        den = _wseg_sum_rows(ones_n, ex, src2, dst2)[:, 0]
        out = _wseg_sum_rows(v_, ex, src2, dst2) / (den[:, None] + 1e-16)
        return relu(out) if active else out

    gcn_enc1 = gcn(x, p['Wg1'], True)
    gcn_enc2 = gcn((1 - A) * gcn_enc1 + A * e1, p['Wg2'], True)
    gcn_enc3 = gcn((1 - A) * gcn_enc2 + A * e2, p['Wg3'], True)
    z_gcn = gcn((1 - A) * gcn_enc3 + A * e3, p['Wg4'], False)

    g1 = gt(x, p['Wq1'], p['Wk1'], p['Wv1'], True)
    g2 = gt((1 - A) * g1 + A * e1, p['Wq2'], p['Wk2'], p['Wv2'], True)
    g3 = gt((1 - A) * g2 + A * e2, p['Wq3'], p['Wk3'], p['Wv3'], True)
    z_graph = gcn((1 - A) * g3 + A * e3, p['Wg4'], False)

    z_i = ALPHA * z_gcn + BETA * z_ae + p['gamma'] * z_graph
    z_l, edge_gcn_hat = _big(adj, z_i, z_gcn)

    gd1 = gcn(z_gcn, p['Wg5'], True)
    gd2 = gcn(gd1, p['Wg6'], True)
    gd3 = gcn(gd2, p['Wg7'], True)
    z_gcn_hat = gcn(gd3, p['Wg8'], True)

    td1 = gt(z_graph, p['Wq5'], p['Wk5'], p['Wv5'], True)
    td2 = gt(td1, p['Wq6'], p['Wk6'], p['Wv6'], True)
    td3 = gt(td2, p['Wq7'], p['Wk7'], p['Wv7'], True)
    z_graph_hat = gcn(td3, p['Wg8'], True)

    q = _tdist(z_l, p['cluster'])
    q1 = _tdist(z_ae, p['cluster'])

    return (x_bar, z_gcn_hat, z_graph_hat, edge_gcn_hat, z_ae, q, q1, z_l)
